# baseline (speedup 1.0000x reference)
import functools

import jax
import jax.numpy as jnp
from jax import lax
from jax.experimental import pallas as pl
from jax.experimental.pallas import tpu as pltpu
from jax.experimental.pallas import tpu_sc as plsc

NN = 50000
EE = 800000
LBL = 47
NP = 50176
E2 = 819200
PB = 50048
CH = 3200
ZB = 3136
LW = 24
S_SP = NP * LW
S_DUMMY = NN * LW
FB = 256

_mesh = plsc.VectorSubcoreMesh(core_axis_name="c", subcore_axis_name="s")


def _fill(ref, n, val, dtype):
    v = jnp.full((16,), val, dtype=dtype)

    def body(i, _):
        ref[pl.ds(i * 16, 16)] = v
        return 0

    lax.fori_loop(0, n // 16, body, 0)


def _fill2d(ref, nrows, w, val, dtype):
    v = jnp.full((16,), val, dtype=dtype)

    def row(r, _):
        def col(j, _):
            ref[r, pl.ds(j * 16, 16)] = v
            return 0

        lax.fori_loop(0, w // 16, col, 0)
        return 0

    lax.fori_loop(0, nrows, row, 0)


def _deg_body(src_hbm, dst_hbm, out_hbm, idxbuf, ones, zbuf, acc_o, acc_i):
    c = lax.axis_index("c")
    s = lax.axis_index("s")
    _fill(zbuf, ZB, 0.0, jnp.float32)
    _fill(ones, CH, 1.0, jnp.float32)
    pltpu.sync_copy(zbuf, acc_o.at[pl.ds(s * ZB, ZB)])
    pltpu.sync_copy(zbuf, acc_i.at[pl.ds(s * ZB, ZB)])
    plsc.subcore_barrier()
    base = (c * 16 + s) * (E2 // 32)

    def chunk(k, _):
        pltpu.sync_copy(src_hbm.at[pl.ds(base + k * CH, CH)], idxbuf)
        pltpu.sync_copy(ones, acc_o.at[idxbuf], add=True)
        pltpu.sync_copy(dst_hbm.at[pl.ds(base + k * CH, CH)], idxbuf)
        pltpu.sync_copy(ones, acc_i.at[idxbuf], add=True)
        return 0

    lax.fori_loop(0, E2 // 32 // CH, chunk, 0)
    plsc.subcore_barrier()
    pltpu.sync_copy(acc_o.at[pl.ds(s * ZB, ZB)], zbuf)
    pltpu.sync_copy(zbuf, out_hbm.at[pl.ds((c * 2 + 0) * NP + s * ZB, ZB)])
    pltpu.sync_copy(acc_i.at[pl.ds(s * ZB, ZB)], zbuf)
    pltpu.sync_copy(zbuf, out_hbm.at[pl.ds((c * 2 + 1) * NP + s * ZB, ZB)])


_deg_call = pl.kernel(
    _deg_body,
    out_type=jax.ShapeDtypeStruct((4 * NP,), jnp.float32),
    mesh=_mesh,
    scratch_types=[
        pltpu.VMEM((CH,), jnp.int32),
        pltpu.VMEM((CH,), jnp.float32),
        pltpu.VMEM((ZB,), jnp.float32),
        pltpu.VMEM_SHARED((NP,), jnp.float32),
        pltpu.VMEM_SHARED((NP,), jnp.float32),
    ],
)


def _s_body(src_hbm, dst_hbm, ns_hbm, lab_hbm, out_hbm,
            srcbuf, dstbuf, normbuf, labbuf, idxacc, valacc, zbuf, sem, s_sp):
    c = lax.axis_index("c")
    s = lax.axis_index("s")
    _fill(zbuf, ZB, 0.0, jnp.float32)
    for k in range(S_SP // 16 // ZB):
        pltpu.sync_copy(zbuf, s_sp.at[pl.ds(s * (S_SP // 16) + k * ZB, ZB)])
    plsc.subcore_barrier()
    base = s * (E2 // 16)
    lab_lo = c * LW
    lab_hi = LW + c * (LBL - LW)

    def chunk(k, _):
        pltpu.sync_copy(src_hbm.at[pl.ds(base + k * CH, CH)], srcbuf)
        pltpu.sync_copy(dst_hbm.at[pl.ds(base + k * CH, CH)], dstbuf)
        pltpu.async_copy(ns_hbm.at[srcbuf], normbuf, sem).wait()
        pltpu.async_copy(lab_hbm.at[srcbuf], labbuf, sem).wait()

        def vec(i, _):
            dstv = dstbuf[pl.ds(i * 16, 16)]
            normv = normbuf[pl.ds(i * 16, 16)]
            labi = labbuf[pl.ds(i * 16, 16)]
            inr = (labi >= lab_lo) & (labi < lab_hi) & (dstv < NN)
            flat = dstv * LW + (labi - lab_lo)
            dummy = S_DUMMY + (dstv & 2047)
            idxacc[pl.ds(i * 16, 16)] = jnp.where(inr, flat, dummy)
            valacc[pl.ds(i * 16, 16)] = jnp.where(inr, normv,
                                                  jnp.zeros((16,), jnp.float32))
            return 0

        lax.fori_loop(0, CH // 16, vec, 0)
        pltpu.sync_copy(valacc, s_sp.at[idxacc], add=True)
        return 0

    lax.fori_loop(0, E2 // 16 // CH, chunk, 0)
    plsc.subcore_barrier()

    def cpout(k, _):
        off = s * (S_SP // 16) + k * ZB
        pltpu.sync_copy(s_sp.at[pl.ds(off, ZB)], zbuf)
        pltpu.sync_copy(zbuf, out_hbm.at[pl.ds(c * S_SP + off, ZB)])
        return 0

    lax.fori_loop(0, S_SP // 16 // ZB, cpout, 0)


_s_call = pl.kernel(
    _s_body,
    out_type=jax.ShapeDtypeStruct((2 * S_SP,), jnp.float32),
    mesh=_mesh,
    scratch_types=[
        pltpu.VMEM((CH,), jnp.int32),
        pltpu.VMEM((CH,), jnp.int32),
        pltpu.VMEM((CH,), jnp.float32),
        pltpu.VMEM((CH,), jnp.int32),
        pltpu.VMEM((CH,), jnp.int32),
        pltpu.VMEM((CH,), jnp.float32),
        pltpu.VMEM((ZB,), jnp.float32),
        pltpu.SemaphoreType.DMA,
        pltpu.VMEM_SHARED((S_SP,), jnp.float32),
    ],
)


PW = 128
NRG = 4
RNG = NP // NRG
GRP = 64
CHP = 1024


def _prop_body(src_hbm, dst_hbm, m_hbm, agg_hbm,
               srcbuf, dstbuf, idx128, loc128, rows, zbuf2, sem, sem2,
               agg_sp):
    ZR = ZB // PW
    rows_per_tile = (RNG + 64) // 16
    c = lax.axis_index("c")
    s = lax.axis_index("s")
    _fill2d(zbuf2, ZR, PW, 0.0, jnp.float32)
    base = s * (E2 // 16)

    for r in range(NRG // 2):
        lo = (c * (NRG // 2) + r) * RNG
        row0 = s * rows_per_tile
        for k in range(rows_per_tile // ZR):
            pltpu.sync_copy(zbuf2, agg_sp.at[pl.ds(row0 + k * ZR, ZR)])
        if rows_per_tile % ZR:
            pltpu.sync_copy(
                zbuf2.at[pl.ds(0, rows_per_tile % ZR)],
                agg_sp.at[pl.ds(row0 + (rows_per_tile // ZR) * ZR,
                                rows_per_tile % ZR)])
        plsc.subcore_barrier()

        NG = CHP // GRP

        def chunk(k, _):
            pltpu.sync_copy(src_hbm.at[pl.ds(base + k * CHP, CHP)], srcbuf)
            pltpu.sync_copy(dst_hbm.at[pl.ds(base + k * CHP, CHP)], dstbuf)

            def vec(i, _):
                off = i * 16
                dstv = dstbuf[pl.ds(off, 16)]
                inr = (dstv >= lo) & (dstv < lo + RNG)
                loc = jnp.where(inr, dstv - lo, RNG + (dstv & 63))
                idx128[i // (GRP // 16), pl.ds((i % (GRP // 16)) * 16, 16)] \
                    = srcbuf[pl.ds(off, 16)]
                loc128[i // (GRP // 16), pl.ds((i % (GRP // 16)) * 16, 16)] \
                    = loc
                return 0

            lax.fori_loop(0, CHP // 16, vec, 0)
            sems = [sem, sem2]
            descs = [None, None]
            descs[0] = pltpu.async_copy(m_hbm.at[idx128.at[0]],
                                        rows.at[0], sems[0])
            for g in range(NG):
                b = g % 2
                descs[b].wait()
                if g + 1 < NG:
                    nb = (g + 1) % 2
                    descs[nb] = pltpu.async_copy(
                        m_hbm.at[idx128.at[g + 1]], rows.at[nb], sems[nb])
                pltpu.sync_copy(rows.at[b], agg_sp.at[loc128.at[g]],
                                add=True)
            return 0

        lax.fori_loop(0, E2 // 16 // CHP, chunk, 0)
        plsc.subcore_barrier()
        out_rows = RNG // 16
        for k in range(out_rows // GRP):
            pltpu.sync_copy(agg_sp.at[pl.ds(s * out_rows + k * GRP, GRP)],
                            rows.at[0])
            pltpu.sync_copy(
                rows.at[0],
                agg_hbm.at[pl.ds(lo + s * out_rows + k * GRP, GRP)])
        orem = out_rows % GRP
        if orem:
            pltpu.sync_copy(
                agg_sp.at[pl.ds(s * out_rows + (out_rows // GRP) * GRP,
                                orem)],
                rows.at[0, pl.ds(0, orem)])
            pltpu.sync_copy(
                rows.at[0, pl.ds(0, orem)],
                agg_hbm.at[pl.ds(lo + s * out_rows + (out_rows // GRP)
                                 * GRP, orem)])
        plsc.subcore_barrier()


_prop = pl.kernel(
    _prop_body,
    out_type=jax.ShapeDtypeStruct((NP, PW), jnp.float32),
    mesh=_mesh,
    scratch_types=[
        pltpu.VMEM((CHP,), jnp.int32),
        pltpu.VMEM((CHP,), jnp.int32),
        pltpu.VMEM((CHP // GRP, GRP), jnp.int32),
        pltpu.VMEM((CHP // GRP, GRP), jnp.int32),
        pltpu.VMEM((2, GRP, PW), jnp.float32),
        pltpu.VMEM((ZB // PW, PW), jnp.float32),
        pltpu.SemaphoreType.DMA,
        pltpu.SemaphoreType.DMA,
        pltpu.VMEM_SHARED((RNG + 64, PW), jnp.float32),
    ],
)


def _norm_body(deg_ref, ns_ref, nd_ref, ns1_ref):
    d = deg_ref[...]
    deg_o = d[0, 0] + d[1, 0]
    deg_i = d[0, 1] + d[1, 1]
    ns = lax.rsqrt(jnp.maximum(deg_o, 1.0))
    nd = lax.rsqrt(jnp.maximum(deg_i, 1.0))
    ns_ref[...] = ns[:, None]
    nd_ref[...] = nd[:, None]
    ns1_ref[...] = ns


def _t1_body(emb_ref, w1_ref, t1_ref):
    t1_ref[...] = jnp.dot(emb_ref[...], w1_ref[...],
                          preferred_element_type=jnp.float32)


def _layer1_body(s0_ref, s1_ref, t1_ref, w2_ref, ns_ref, nd_ref,
                 b1_ref, m2_ref):
    t1 = t1_ref[...]
    agg = (jnp.dot(s0_ref[...], t1[0:LW], preferred_element_type=jnp.float32)
           + jnp.dot(s1_ref[...], t1[LW:2 * LW],
                     preferred_element_type=jnp.float32))
    hn = jax.nn.relu(agg * nd_ref[...] + b1_ref[...]) * ns_ref[...]
    m2_ref[...] = jnp.dot(hn, w2_ref[...], preferred_element_type=jnp.float32)


def _layer2_body(agg_ref, w3_ref, ns_ref, nd_ref, b2_ref, m3_ref):
    h = jax.nn.relu(agg_ref[...][:, :100] * nd_ref[...] + b2_ref[...])
    m3_ref[...] = jnp.dot(h * ns_ref[...], w3_ref[...],
                          preferred_element_type=jnp.float32)


def _layer3_body(agg_ref, nd_ref, b3_ref, out_ref):
    out_ref[...] = agg_ref[...][:, :LBL] * nd_ref[...] + b3_ref[...]


_BLK = 512
_GRID = NP // _BLK


def _col_spec(w):
    return pl.BlockSpec((_BLK, w), lambda i: (i, 0))


def _full_spec(r, w):
    return pl.BlockSpec((r, w), lambda i: (0, 0))


def kernel(dep_labels, edge_index, emb, W1, b1, W2, b2, W3, b3):
    f32 = jnp.float32
    pad_ids = PB + (jnp.arange(E2 - EE, dtype=jnp.int32) % 128)
    src_p = jnp.concatenate([edge_index[0], pad_ids])
    dst_p = jnp.concatenate([edge_index[1], pad_ids])
    lab1d = jnp.pad(dep_labels, (0, NP - NN))
    embp = jnp.pad(emb, ((0, 1), (0, 14)))
    W1p = jnp.pad(W1, ((0, 14), (0, 12)))
    W2p = jnp.pad(W2, ((0, 12), (0, 28)))
    W3p = jnp.pad(W3, ((0, 0), (0, 81)))
    b1p = jnp.pad(b1, (0, 12))[None, :]
    b2p = b2[None, :]
    b3p = b3[None, :]

    degpart = _deg_call(src_p, dst_p).reshape(2, 2, NP)

    ns2d, nd2d, ns1d = pl.pallas_call(
        _norm_body,
        grid=(_GRID,),
        in_specs=[pl.BlockSpec((2, 2, _BLK), lambda i: (0, 0, i))],
        out_specs=[_col_spec(1), _col_spec(1),
                   pl.BlockSpec((_BLK,), lambda i: (i,))],
        out_shape=[jax.ShapeDtypeStruct((NP, 1), f32),
                   jax.ShapeDtypeStruct((NP, 1), f32),
                   jax.ShapeDtypeStruct((NP,), f32)],
    )(degpart)

    t1 = pl.pallas_call(
        _t1_body,
        in_specs=[pl.BlockSpec((48, 64), lambda: (0, 0)),
                  pl.BlockSpec((64, 112), lambda: (0, 0))],
        out_specs=pl.BlockSpec((48, 112), lambda: (0, 0)),
        out_shape=jax.ShapeDtypeStruct((48, 112), f32),
    )(embp, W1p)

    s_flat = _s_call(src_p, dst_p, ns1d, lab1d).reshape(2, NP, LW)
    s0 = s_flat[0]
    s1 = s_flat[1]

    m2 = pl.pallas_call(
        _layer1_body,
        grid=(_GRID,),
        in_specs=[_col_spec(LW), _col_spec(LW), _full_spec(48, 112),
                  _full_spec(112, PW), _col_spec(1),
                  _col_spec(1), _full_spec(1, 112)],
        out_specs=_col_spec(PW),
        out_shape=jax.ShapeDtypeStruct((NP, PW), f32),
    )(s0, s1, t1, W2p, ns2d, nd2d, b1p)

    agg2 = _prop(src_p, dst_p, m2)

    m3 = pl.pallas_call(
        _layer2_body,
        grid=(_GRID,),
        in_specs=[_col_spec(PW), _full_spec(100, PW),
                  _col_spec(1), _col_spec(1), _full_spec(1, 100)],
        out_specs=_col_spec(PW),
        out_shape=jax.ShapeDtypeStruct((NP, PW), f32),
    )(agg2, W3p, ns2d, nd2d, b2p)

    agg3 = _prop(src_p, dst_p, m3)

    out = pl.pallas_call(
        _layer3_body,
        grid=(_GRID,),
        in_specs=[_col_spec(PW), _col_spec(1), _full_spec(1, LBL)],
        out_specs=_col_spec(LBL),
        out_shape=jax.ShapeDtypeStruct((NP, LBL), f32),
    )(agg3, nd2d, b3p)

    return out[:NN]


# speedup vs baseline: 1.0629x; 1.0629x over previous
import functools

import jax
import jax.numpy as jnp
from jax import lax
from jax.experimental import pallas as pl
from jax.experimental.pallas import tpu as pltpu
from jax.experimental.pallas import tpu_sc as plsc

NN = 50000
EE = 800000
LBL = 47
NP = 50176
E2 = 819200
PB = 50048
CH = 3200
ZB = 3136
LW = 24
S_SP = NP * LW
S_DUMMY = NN * LW
FB = 256

_mesh = plsc.VectorSubcoreMesh(core_axis_name="c", subcore_axis_name="s")


def _fill(ref, n, val, dtype):
    v = jnp.full((16,), val, dtype=dtype)

    def body(i, _):
        ref[pl.ds(i * 16, 16)] = v
        return 0

    lax.fori_loop(0, n // 16, body, 0)


def _fill2d(ref, nrows, w, val, dtype):
    v = jnp.full((16,), val, dtype=dtype)

    def row(r, _):
        def col(j, _):
            ref[r, pl.ds(j * 16, 16)] = v
            return 0

        lax.fori_loop(0, w // 16, col, 0)
        return 0

    lax.fori_loop(0, nrows, row, 0)


def _deg_body(src_hbm, dst_hbm, out_hbm, idxbuf, ones, zbuf, acc_o, acc_i):
    c = lax.axis_index("c")
    s = lax.axis_index("s")
    _fill(zbuf, ZB, 0.0, jnp.float32)
    _fill(ones, CH, 1.0, jnp.float32)
    pltpu.sync_copy(zbuf, acc_o.at[pl.ds(s * ZB, ZB)])
    pltpu.sync_copy(zbuf, acc_i.at[pl.ds(s * ZB, ZB)])
    plsc.subcore_barrier()
    base = (c * 16 + s) * (E2 // 32)

    def chunk(k, _):
        pltpu.sync_copy(src_hbm.at[pl.ds(base + k * CH, CH)], idxbuf)
        pltpu.sync_copy(ones, acc_o.at[idxbuf], add=True)
        pltpu.sync_copy(dst_hbm.at[pl.ds(base + k * CH, CH)], idxbuf)
        pltpu.sync_copy(ones, acc_i.at[idxbuf], add=True)
        return 0

    lax.fori_loop(0, E2 // 32 // CH, chunk, 0)
    plsc.subcore_barrier()
    pltpu.sync_copy(acc_o.at[pl.ds(s * ZB, ZB)], zbuf)
    pltpu.sync_copy(zbuf, out_hbm.at[pl.ds((c * 2 + 0) * NP + s * ZB, ZB)])
    pltpu.sync_copy(acc_i.at[pl.ds(s * ZB, ZB)], zbuf)
    pltpu.sync_copy(zbuf, out_hbm.at[pl.ds((c * 2 + 1) * NP + s * ZB, ZB)])


_deg_call = pl.kernel(
    _deg_body,
    out_type=jax.ShapeDtypeStruct((4 * NP,), jnp.float32),
    mesh=_mesh,
    scratch_types=[
        pltpu.VMEM((CH,), jnp.int32),
        pltpu.VMEM((CH,), jnp.float32),
        pltpu.VMEM((ZB,), jnp.float32),
        pltpu.VMEM_SHARED((NP,), jnp.float32),
        pltpu.VMEM_SHARED((NP,), jnp.float32),
    ],
)


def _s_body(src_hbm, dst_hbm, pk_hbm, out_hbm,
            srcbuf, dstbuf, pkbuf, idxacc, valacc, zbuf, sem, s_sp):
    c = lax.axis_index("c")
    s = lax.axis_index("s")
    _fill(zbuf, ZB, 0.0, jnp.float32)
    for k in range(S_SP // 16 // ZB):
        pltpu.sync_copy(zbuf, s_sp.at[pl.ds(s * (S_SP // 16) + k * ZB, ZB)])
    plsc.subcore_barrier()
    base = s * (E2 // 16)
    lab_lo = c * LW
    lab_hi = LW + c * (LBL - LW)

    def chunk(k, _):
        pltpu.sync_copy(src_hbm.at[pl.ds(base + k * CH, CH)], srcbuf)
        pltpu.sync_copy(dst_hbm.at[pl.ds(base + k * CH, CH)], dstbuf)
        pltpu.async_copy(pk_hbm.at[srcbuf], pkbuf, sem).wait()

        def vec(i, _):
            dstv = dstbuf[pl.ds(i * 16, 16)]
            pk = pkbuf[pl.ds(i * 16, 16)]
            labi = lax.shift_right_logical(pk.astype(jnp.int32), 1)
            normv = pk - (labi * 2).astype(jnp.float32)
            inr = (labi >= lab_lo) & (labi < lab_hi) & (dstv < NN)
            flat = dstv * LW + (labi - lab_lo)
            dummy = S_DUMMY + (dstv & 2047)
            idxacc[pl.ds(i * 16, 16)] = jnp.where(inr, flat, dummy)
            valacc[pl.ds(i * 16, 16)] = jnp.where(inr, normv,
                                                  jnp.zeros((16,), jnp.float32))
            return 0

        lax.fori_loop(0, CH // 16, vec, 0)
        pltpu.sync_copy(valacc, s_sp.at[idxacc], add=True)
        return 0

    lax.fori_loop(0, E2 // 16 // CH, chunk, 0)
    plsc.subcore_barrier()

    def cpout(k, _):
        off = s * (S_SP // 16) + k * ZB
        pltpu.sync_copy(s_sp.at[pl.ds(off, ZB)], zbuf)
        pltpu.sync_copy(zbuf, out_hbm.at[pl.ds(c * S_SP + off, ZB)])
        return 0

    lax.fori_loop(0, S_SP // 16 // ZB, cpout, 0)


_s_call = pl.kernel(
    _s_body,
    out_type=jax.ShapeDtypeStruct((2 * S_SP,), jnp.float32),
    mesh=_mesh,
    scratch_types=[
        pltpu.VMEM((CH,), jnp.int32),
        pltpu.VMEM((CH,), jnp.int32),
        pltpu.VMEM((CH,), jnp.float32),
        pltpu.VMEM((CH,), jnp.int32),
        pltpu.VMEM((CH,), jnp.float32),
        pltpu.VMEM((ZB,), jnp.float32),
        pltpu.SemaphoreType.DMA,
        pltpu.VMEM_SHARED((S_SP,), jnp.float32),
    ],
)


PW = 128
NRG = 4
RNG = NP // NRG
GRP = 128
CHP = 1024


def _prop_body(src_hbm, dst_hbm, m_hbm, agg_hbm,
               srcbuf, dstbuf, idx128, loc128, rows, zbuf2, sem, sem2,
               agg_sp):
    ZR = ZB // PW
    rows_per_tile = (RNG + 64) // 16
    c = lax.axis_index("c")
    s = lax.axis_index("s")
    _fill2d(zbuf2, ZR, PW, 0.0, jnp.float32)
    base = s * (E2 // 16)

    for r in range(NRG // 2):
        lo = (c * (NRG // 2) + r) * RNG
        row0 = s * rows_per_tile
        for k in range(rows_per_tile // ZR):
            pltpu.sync_copy(zbuf2, agg_sp.at[pl.ds(row0 + k * ZR, ZR)])
        if rows_per_tile % ZR:
            pltpu.sync_copy(
                zbuf2.at[pl.ds(0, rows_per_tile % ZR)],
                agg_sp.at[pl.ds(row0 + (rows_per_tile // ZR) * ZR,
                                rows_per_tile % ZR)])
        plsc.subcore_barrier()

        NG = CHP // GRP

        def chunk(k, _):
            pltpu.sync_copy(src_hbm.at[pl.ds(base + k * CHP, CHP)], srcbuf)
            pltpu.sync_copy(dst_hbm.at[pl.ds(base + k * CHP, CHP)], dstbuf)

            def vec(i, _):
                off = i * 16
                dstv = dstbuf[pl.ds(off, 16)]
                inr = (dstv >= lo) & (dstv < lo + RNG)
                loc = jnp.where(inr, dstv - lo, RNG + (dstv & 63))
                idx128[i // (GRP // 16), pl.ds((i % (GRP // 16)) * 16, 16)] \
                    = srcbuf[pl.ds(off, 16)]
                loc128[i // (GRP // 16), pl.ds((i % (GRP // 16)) * 16, 16)] \
                    = loc
                return 0

            lax.fori_loop(0, CHP // 16, vec, 0)
            for g in range(NG):
                pltpu.async_copy(m_hbm.at[idx128.at[g]], rows.at[0],
                                 sem).wait()
                pltpu.sync_copy(rows.at[0], agg_sp.at[loc128.at[g]],
                                add=True)
            return 0

        lax.fori_loop(0, E2 // 16 // CHP, chunk, 0)
        plsc.subcore_barrier()
        out_rows = RNG // 16
        for k in range(out_rows // GRP):
            pltpu.sync_copy(agg_sp.at[pl.ds(s * out_rows + k * GRP, GRP)],
                            rows.at[0])
            pltpu.sync_copy(
                rows.at[0],
                agg_hbm.at[pl.ds(lo + s * out_rows + k * GRP, GRP)])
        orem = out_rows % GRP
        if orem:
            pltpu.sync_copy(
                agg_sp.at[pl.ds(s * out_rows + (out_rows // GRP) * GRP,
                                orem)],
                rows.at[0, pl.ds(0, orem)])
            pltpu.sync_copy(
                rows.at[0, pl.ds(0, orem)],
                agg_hbm.at[pl.ds(lo + s * out_rows + (out_rows // GRP)
                                 * GRP, orem)])
        plsc.subcore_barrier()


_prop = pl.kernel(
    _prop_body,
    out_type=jax.ShapeDtypeStruct((NP, PW), jnp.float32),
    mesh=_mesh,
    scratch_types=[
        pltpu.VMEM((CHP,), jnp.int32),
        pltpu.VMEM((CHP,), jnp.int32),
        pltpu.VMEM((CHP // GRP, GRP), jnp.int32),
        pltpu.VMEM((CHP // GRP, GRP), jnp.int32),
        pltpu.VMEM((1, GRP, PW), jnp.float32),
        pltpu.VMEM((ZB // PW, PW), jnp.float32),
        pltpu.SemaphoreType.DMA,
        pltpu.SemaphoreType.DMA,
        pltpu.VMEM_SHARED((RNG + 64, PW), jnp.float32),
    ],
)


def _norm_body(deg_ref, lab_ref, ns_ref, nd_ref, pk_ref):
    d = deg_ref[...]
    deg_o = d[0, 0] + d[1, 0]
    deg_i = d[0, 1] + d[1, 1]
    ns = lax.rsqrt(jnp.maximum(deg_o, 1.0))
    nd = lax.rsqrt(jnp.maximum(deg_i, 1.0))
    ns_ref[...] = ns[:, None]
    nd_ref[...] = nd[:, None]
    pk_ref[...] = ns + 2.0 * lab_ref[..., 0]


def _t1_body(emb_ref, w1_ref, t1_ref):
    t1_ref[...] = jnp.dot(emb_ref[...], w1_ref[...],
                          preferred_element_type=jnp.float32)


def _layer1_body(s0_ref, s1_ref, t1_ref, w2_ref, ns_ref, nd_ref,
                 b1_ref, m2_ref):
    t1 = t1_ref[...]
    agg = (jnp.dot(s0_ref[...], t1[0:LW], preferred_element_type=jnp.float32)
           + jnp.dot(s1_ref[...], t1[LW:2 * LW],
                     preferred_element_type=jnp.float32))
    hn = jax.nn.relu(agg * nd_ref[...] + b1_ref[...]) * ns_ref[...]
    m2_ref[...] = jnp.dot(hn, w2_ref[...], preferred_element_type=jnp.float32)


def _layer2_body(agg_ref, w3_ref, ns_ref, nd_ref, b2_ref, m3_ref):
    h = jax.nn.relu(agg_ref[...][:, :100] * nd_ref[...] + b2_ref[...])
    m3_ref[...] = jnp.dot(h * ns_ref[...], w3_ref[...],
                          preferred_element_type=jnp.float32)


def _layer3_body(agg_ref, nd_ref, b3_ref, out_ref):
    out_ref[...] = agg_ref[...][:, :LBL] * nd_ref[...] + b3_ref[...]


_BLK = 512
_GRID = NP // _BLK


def _col_spec(w):
    return pl.BlockSpec((_BLK, w), lambda i: (i, 0))


def _full_spec(r, w):
    return pl.BlockSpec((r, w), lambda i: (0, 0))


def kernel(dep_labels, edge_index, emb, W1, b1, W2, b2, W3, b3):
    f32 = jnp.float32
    pad_ids = PB + (jnp.arange(E2 - EE, dtype=jnp.int32) % 128)
    src_p = jnp.concatenate([edge_index[0], pad_ids])
    dst_p = jnp.concatenate([edge_index[1], pad_ids])
    lab1d = jnp.pad(dep_labels, (0, NP - NN))
    embp = jnp.pad(emb, ((0, 1), (0, 14)))
    W1p = jnp.pad(W1, ((0, 14), (0, 12)))
    W2p = jnp.pad(W2, ((0, 12), (0, 28)))
    W3p = jnp.pad(W3, ((0, 0), (0, 81)))
    b1p = jnp.pad(b1, (0, 12))[None, :]
    b2p = b2[None, :]
    b3p = b3[None, :]

    degpart = _deg_call(src_p, dst_p).reshape(2, 2, NP)

    ns2d, nd2d, packed = pl.pallas_call(
        _norm_body,
        grid=(_GRID,),
        in_specs=[pl.BlockSpec((2, 2, _BLK), lambda i: (0, 0, i)),
                  _col_spec(1)],
        out_specs=[_col_spec(1), _col_spec(1),
                   pl.BlockSpec((_BLK,), lambda i: (i,))],
        out_shape=[jax.ShapeDtypeStruct((NP, 1), f32),
                   jax.ShapeDtypeStruct((NP, 1), f32),
                   jax.ShapeDtypeStruct((NP,), f32)],
    )(degpart, lab1d.astype(f32)[:, None])

    t1 = pl.pallas_call(
        _t1_body,
        in_specs=[pl.BlockSpec((48, 64), lambda: (0, 0)),
                  pl.BlockSpec((64, 112), lambda: (0, 0))],
        out_specs=pl.BlockSpec((48, 112), lambda: (0, 0)),
        out_shape=jax.ShapeDtypeStruct((48, 112), f32),
    )(embp, W1p)

    s_flat = _s_call(src_p, dst_p, packed).reshape(2, NP, LW)
    s0 = s_flat[0]
    s1 = s_flat[1]

    m2 = pl.pallas_call(
        _layer1_body,
        grid=(_GRID,),
        in_specs=[_col_spec(LW), _col_spec(LW), _full_spec(48, 112),
                  _full_spec(112, PW), _col_spec(1),
                  _col_spec(1), _full_spec(1, 112)],
        out_specs=_col_spec(PW),
        out_shape=jax.ShapeDtypeStruct((NP, PW), f32),
    )(s0, s1, t1, W2p, ns2d, nd2d, b1p)

    agg2 = _prop(src_p, dst_p, m2)

    m3 = pl.pallas_call(
        _layer2_body,
        grid=(_GRID,),
        in_specs=[_col_spec(PW), _full_spec(100, PW),
                  _col_spec(1), _col_spec(1), _full_spec(1, 100)],
        out_specs=_col_spec(PW),
        out_shape=jax.ShapeDtypeStruct((NP, PW), f32),
    )(agg2, W3p, ns2d, nd2d, b2p)

    agg3 = _prop(src_p, dst_p, m3)

    out = pl.pallas_call(
        _layer3_body,
        grid=(_GRID,),
        in_specs=[_col_spec(PW), _col_spec(1), _full_spec(1, LBL)],
        out_specs=_col_spec(LBL),
        out_shape=jax.ShapeDtypeStruct((NP, LBL), f32),
    )(agg3, nd2d, b3p)

    return out[:NN]


# speedup vs baseline: 1.0920x; 1.0274x over previous
import functools

import jax
import jax.numpy as jnp
from jax import lax
from jax.experimental import pallas as pl
from jax.experimental.pallas import tpu as pltpu
from jax.experimental.pallas import tpu_sc as plsc

NN = 50000
EE = 800000
LBL = 47
NP = 50176
E2 = 819200
PB = 50048
CH = 3200
ZB = 3136
LW = 24
S_SP = NP * LW
S_DUMMY = NN * LW
FB = 256

_mesh = plsc.VectorSubcoreMesh(core_axis_name="c", subcore_axis_name="s")


def _fill(ref, n, val, dtype):
    v = jnp.full((16,), val, dtype=dtype)

    def body(i, _):
        ref[pl.ds(i * 16, 16)] = v
        return 0

    lax.fori_loop(0, n // 16, body, 0)


def _fill2d(ref, nrows, w, val, dtype):
    v = jnp.full((16,), val, dtype=dtype)

    def row(r, _):
        def col(j, _):
            ref[r, pl.ds(j * 16, 16)] = v
            return 0

        lax.fori_loop(0, w // 16, col, 0)
        return 0

    lax.fori_loop(0, nrows, row, 0)


def _deg_body(src_hbm, dst_hbm, out_hbm, idxbuf, ones, zbuf, acc_o, acc_i):
    c = lax.axis_index("c")
    s = lax.axis_index("s")
    _fill(zbuf, ZB, 0.0, jnp.float32)
    _fill(ones, CH, 1.0, jnp.float32)
    pltpu.sync_copy(zbuf, acc_o.at[pl.ds(s * ZB, ZB)])
    pltpu.sync_copy(zbuf, acc_i.at[pl.ds(s * ZB, ZB)])
    plsc.subcore_barrier()
    base = (c * 16 + s) * (E2 // 32)

    def chunk(k, _):
        pltpu.sync_copy(src_hbm.at[pl.ds(base + k * CH, CH)], idxbuf)
        pltpu.sync_copy(ones, acc_o.at[idxbuf], add=True)
        pltpu.sync_copy(dst_hbm.at[pl.ds(base + k * CH, CH)], idxbuf)
        pltpu.sync_copy(ones, acc_i.at[idxbuf], add=True)
        return 0

    lax.fori_loop(0, E2 // 32 // CH, chunk, 0)
    plsc.subcore_barrier()
    pltpu.sync_copy(acc_o.at[pl.ds(s * ZB, ZB)], zbuf)
    pltpu.sync_copy(zbuf, out_hbm.at[pl.ds((c * 2 + 0) * NP + s * ZB, ZB)])
    pltpu.sync_copy(acc_i.at[pl.ds(s * ZB, ZB)], zbuf)
    pltpu.sync_copy(zbuf, out_hbm.at[pl.ds((c * 2 + 1) * NP + s * ZB, ZB)])


_deg_call = pl.kernel(
    _deg_body,
    out_type=jax.ShapeDtypeStruct((4 * NP,), jnp.float32),
    mesh=_mesh,
    scratch_types=[
        pltpu.VMEM((CH,), jnp.int32),
        pltpu.VMEM((CH,), jnp.float32),
        pltpu.VMEM((ZB,), jnp.float32),
        pltpu.VMEM_SHARED((NP,), jnp.float32),
        pltpu.VMEM_SHARED((NP,), jnp.float32),
    ],
)


def _s_body(src_hbm, dst_hbm, pk_hbm, out_hbm,
            srcbuf, dstbuf, pkbuf, idxacc, valacc, zbuf, sem, s_sp):
    c = lax.axis_index("c")
    s = lax.axis_index("s")
    _fill(zbuf, ZB, 0.0, jnp.float32)
    for k in range(S_SP // 16 // ZB):
        pltpu.sync_copy(zbuf, s_sp.at[pl.ds(s * (S_SP // 16) + k * ZB, ZB)])
    plsc.subcore_barrier()
    base = s * (E2 // 16)
    lab_lo = c * LW
    lab_hi = LW + c * (LBL - LW)

    def chunk(k, _):
        pltpu.sync_copy(src_hbm.at[pl.ds(base + k * CH, CH)], srcbuf)
        pltpu.sync_copy(dst_hbm.at[pl.ds(base + k * CH, CH)], dstbuf)
        pltpu.async_copy(pk_hbm.at[srcbuf], pkbuf, sem).wait()

        def vec(i, _):
            dstv = dstbuf[pl.ds(i * 16, 16)]
            pk = pkbuf[pl.ds(i * 16, 16)]
            labi = lax.shift_right_logical(pk.astype(jnp.int32), 1)
            normv = pk - (labi * 2).astype(jnp.float32)
            inr = (labi >= lab_lo) & (labi < lab_hi) & (dstv < NN)
            flat = dstv * LW + (labi - lab_lo)
            dummy = S_DUMMY + (dstv & 2047)
            idxacc[pl.ds(i * 16, 16)] = jnp.where(inr, flat, dummy)
            valacc[pl.ds(i * 16, 16)] = jnp.where(inr, normv,
                                                  jnp.zeros((16,), jnp.float32))
            return 0

        lax.fori_loop(0, CH // 16, vec, 0)
        pltpu.sync_copy(valacc, s_sp.at[idxacc], add=True)
        return 0

    lax.fori_loop(0, E2 // 16 // CH, chunk, 0)
    plsc.subcore_barrier()

    def cpout(k, _):
        off = s * (S_SP // 16) + k * ZB
        pltpu.sync_copy(s_sp.at[pl.ds(off, ZB)], zbuf)
        pltpu.sync_copy(zbuf, out_hbm.at[pl.ds(c * S_SP + off, ZB)])
        return 0

    lax.fori_loop(0, S_SP // 16 // ZB, cpout, 0)


_s_call = pl.kernel(
    _s_body,
    out_type=jax.ShapeDtypeStruct((2 * S_SP,), jnp.float32),
    mesh=_mesh,
    scratch_types=[
        pltpu.VMEM((CH,), jnp.int32),
        pltpu.VMEM((CH,), jnp.int32),
        pltpu.VMEM((CH,), jnp.float32),
        pltpu.VMEM((CH,), jnp.int32),
        pltpu.VMEM((CH,), jnp.float32),
        pltpu.VMEM((ZB,), jnp.float32),
        pltpu.SemaphoreType.DMA,
        pltpu.VMEM_SHARED((S_SP,), jnp.float32),
    ],
)


PW = 128
NRG = 4
RNG = NP // NRG
GRP = 80
CHP = 640


def _prop_body(src_hbm, dst_hbm, m_hbm, agg_hbm,
               srcbuf, dstbuf, idx128, loc128, rows, zbuf2, sem, sem2,
               agg_sp):
    ZR = ZB // PW
    rows_per_tile = (RNG + 64) // 16
    c = lax.axis_index("c")
    s = lax.axis_index("s")
    _fill2d(zbuf2, ZR, PW, 0.0, jnp.float32)
    base = s * (E2 // 16)

    for r in range(NRG // 2):
        lo = (c * (NRG // 2) + r) * RNG
        row0 = s * rows_per_tile
        for k in range(rows_per_tile // ZR):
            pltpu.sync_copy(zbuf2, agg_sp.at[pl.ds(row0 + k * ZR, ZR)])
        if rows_per_tile % ZR:
            pltpu.sync_copy(
                zbuf2.at[pl.ds(0, rows_per_tile % ZR)],
                agg_sp.at[pl.ds(row0 + (rows_per_tile // ZR) * ZR,
                                rows_per_tile % ZR)])
        plsc.subcore_barrier()

        NG = CHP // GRP

        def chunk(k, _):
            pltpu.sync_copy(src_hbm.at[pl.ds(base + k * CHP, CHP)], srcbuf)
            pltpu.sync_copy(dst_hbm.at[pl.ds(base + k * CHP, CHP)], dstbuf)

            def vec(i, _):
                off = i * 16
                dstv = dstbuf[pl.ds(off, 16)]
                inr = (dstv >= lo) & (dstv < lo + RNG)
                loc = jnp.where(inr, dstv - lo, RNG + (dstv & 63))
                idx128[i // (GRP // 16), pl.ds((i % (GRP // 16)) * 16, 16)] \
                    = srcbuf[pl.ds(off, 16)]
                loc128[i // (GRP // 16), pl.ds((i % (GRP // 16)) * 16, 16)] \
                    = loc
                return 0

            lax.fori_loop(0, CHP // 16, vec, 0)
            dg = [None] * NG
            ds_ = [None] * NG
            dg[0] = pltpu.async_copy(m_hbm.at[idx128.at[0]], rows.at[0], sem)
            for g in range(NG):
                b = g % 2
                dg[g].wait()
                if g + 1 < NG:
                    if g >= 1:
                        ds_[g - 1].wait()
                    dg[g + 1] = pltpu.async_copy(
                        m_hbm.at[idx128.at[g + 1]], rows.at[(g + 1) % 2],
                        sem)
                ds_[g] = pltpu.async_copy(rows.at[b],
                                          agg_sp.at[loc128.at[g]], sem2,
                                          add=True)
            ds_[NG - 2].wait()
            ds_[NG - 1].wait()
            return 0

        lax.fori_loop(0, E2 // 16 // CHP, chunk, 0)
        plsc.subcore_barrier()
        out_rows = RNG // 16
        for k in range(out_rows // GRP):
            pltpu.sync_copy(agg_sp.at[pl.ds(s * out_rows + k * GRP, GRP)],
                            rows.at[0])
            pltpu.sync_copy(
                rows.at[0],
                agg_hbm.at[pl.ds(lo + s * out_rows + k * GRP, GRP)])
        orem = out_rows % GRP
        if orem:
            pltpu.sync_copy(
                agg_sp.at[pl.ds(s * out_rows + (out_rows // GRP) * GRP,
                                orem)],
                rows.at[0, pl.ds(0, orem)])
            pltpu.sync_copy(
                rows.at[0, pl.ds(0, orem)],
                agg_hbm.at[pl.ds(lo + s * out_rows + (out_rows // GRP)
                                 * GRP, orem)])
        plsc.subcore_barrier()


_prop = pl.kernel(
    _prop_body,
    out_type=jax.ShapeDtypeStruct((NP, PW), jnp.float32),
    mesh=_mesh,
    scratch_types=[
        pltpu.VMEM((CHP,), jnp.int32),
        pltpu.VMEM((CHP,), jnp.int32),
        pltpu.VMEM((CHP // GRP, GRP), jnp.int32),
        pltpu.VMEM((CHP // GRP, GRP), jnp.int32),
        pltpu.VMEM((2, GRP, PW), jnp.float32),
        pltpu.VMEM((ZB // PW, PW), jnp.float32),
        pltpu.SemaphoreType.DMA,
        pltpu.SemaphoreType.DMA,
        pltpu.VMEM_SHARED((RNG + 64, PW), jnp.float32),
    ],
)


def _norm_body(deg_ref, lab_ref, ns_ref, nd_ref, pk_ref):
    d = deg_ref[...]
    deg_o = d[0, 0] + d[1, 0]
    deg_i = d[0, 1] + d[1, 1]
    ns = lax.rsqrt(jnp.maximum(deg_o, 1.0))
    nd = lax.rsqrt(jnp.maximum(deg_i, 1.0))
    ns_ref[...] = ns[:, None]
    nd_ref[...] = nd[:, None]
    pk_ref[...] = ns + 2.0 * lab_ref[..., 0]


def _t1_body(emb_ref, w1_ref, t1_ref):
    t1_ref[...] = jnp.dot(emb_ref[...], w1_ref[...],
                          preferred_element_type=jnp.float32)


def _layer1_body(s0_ref, s1_ref, t1_ref, w2_ref, ns_ref, nd_ref,
                 b1_ref, m2_ref):
    t1 = t1_ref[...]
    agg = (jnp.dot(s0_ref[...], t1[0:LW], preferred_element_type=jnp.float32)
           + jnp.dot(s1_ref[...], t1[LW:2 * LW],
                     preferred_element_type=jnp.float32))
    hn = jax.nn.relu(agg * nd_ref[...] + b1_ref[...]) * ns_ref[...]
    m2_ref[...] = jnp.dot(hn, w2_ref[...], preferred_element_type=jnp.float32)


def _layer2_body(agg_ref, w3_ref, ns_ref, nd_ref, b2_ref, m3_ref):
    h = jax.nn.relu(agg_ref[...][:, :100] * nd_ref[...] + b2_ref[...])
    m3_ref[...] = jnp.dot(h * ns_ref[...], w3_ref[...],
                          preferred_element_type=jnp.float32)


def _layer3_body(agg_ref, nd_ref, b3_ref, out_ref):
    out_ref[...] = agg_ref[...][:, :LBL] * nd_ref[...] + b3_ref[...]


_BLK = 512
_GRID = NP // _BLK


def _col_spec(w):
    return pl.BlockSpec((_BLK, w), lambda i: (i, 0))


def _full_spec(r, w):
    return pl.BlockSpec((r, w), lambda i: (0, 0))


def kernel(dep_labels, edge_index, emb, W1, b1, W2, b2, W3, b3):
    f32 = jnp.float32
    pad_ids = PB + (jnp.arange(E2 - EE, dtype=jnp.int32) % 128)
    src_p = jnp.concatenate([edge_index[0], pad_ids])
    dst_p = jnp.concatenate([edge_index[1], pad_ids])
    lab1d = jnp.pad(dep_labels, (0, NP - NN))
    embp = jnp.pad(emb, ((0, 1), (0, 14)))
    W1p = jnp.pad(W1, ((0, 14), (0, 12)))
    W2p = jnp.pad(W2, ((0, 12), (0, 28)))
    W3p = jnp.pad(W3, ((0, 0), (0, 81)))
    b1p = jnp.pad(b1, (0, 12))[None, :]
    b2p = b2[None, :]
    b3p = b3[None, :]

    degpart = _deg_call(src_p, dst_p).reshape(2, 2, NP)

    ns2d, nd2d, packed = pl.pallas_call(
        _norm_body,
        grid=(_GRID,),
        in_specs=[pl.BlockSpec((2, 2, _BLK), lambda i: (0, 0, i)),
                  _col_spec(1)],
        out_specs=[_col_spec(1), _col_spec(1),
                   pl.BlockSpec((_BLK,), lambda i: (i,))],
        out_shape=[jax.ShapeDtypeStruct((NP, 1), f32),
                   jax.ShapeDtypeStruct((NP, 1), f32),
                   jax.ShapeDtypeStruct((NP,), f32)],
    )(degpart, lab1d.astype(f32)[:, None])

    t1 = pl.pallas_call(
        _t1_body,
        in_specs=[pl.BlockSpec((48, 64), lambda: (0, 0)),
                  pl.BlockSpec((64, 112), lambda: (0, 0))],
        out_specs=pl.BlockSpec((48, 112), lambda: (0, 0)),
        out_shape=jax.ShapeDtypeStruct((48, 112), f32),
    )(embp, W1p)

    s_flat = _s_call(src_p, dst_p, packed).reshape(2, NP, LW)
    s0 = s_flat[0]
    s1 = s_flat[1]

    m2 = pl.pallas_call(
        _layer1_body,
        grid=(_GRID,),
        in_specs=[_col_spec(LW), _col_spec(LW), _full_spec(48, 112),
                  _full_spec(112, PW), _col_spec(1),
                  _col_spec(1), _full_spec(1, 112)],
        out_specs=_col_spec(PW),
        out_shape=jax.ShapeDtypeStruct((NP, PW), f32),
    )(s0, s1, t1, W2p, ns2d, nd2d, b1p)

    agg2 = _prop(src_p, dst_p, m2)

    m3 = pl.pallas_call(
        _layer2_body,
        grid=(_GRID,),
        in_specs=[_col_spec(PW), _full_spec(100, PW),
                  _col_spec(1), _col_spec(1), _full_spec(1, 100)],
        out_specs=_col_spec(PW),
        out_shape=jax.ShapeDtypeStruct((NP, PW), f32),
    )(agg2, W3p, ns2d, nd2d, b2p)

    agg3 = _prop(src_p, dst_p, m3)

    out = pl.pallas_call(
        _layer3_body,
        grid=(_GRID,),
        in_specs=[_col_spec(PW), _col_spec(1), _full_spec(1, LBL)],
        out_specs=_col_spec(LBL),
        out_shape=jax.ShapeDtypeStruct((NP, LBL), f32),
    )(agg3, nd2d, b3p)

    return out[:NN]


# speedup vs baseline: 1.1483x; 1.0516x over previous
import functools

import jax
import jax.numpy as jnp
from jax import lax
from jax.experimental import pallas as pl
from jax.experimental.pallas import tpu as pltpu
from jax.experimental.pallas import tpu_sc as plsc

NN = 50000
EE = 800000
LBL = 47
NP = 50176
E2 = 819200
PB = 50048
CH = 3200
ZB = 3136
LW = 24
S_SP = NP * LW
S_DUMMY = NN * LW
FB = 256

_mesh = plsc.VectorSubcoreMesh(core_axis_name="c", subcore_axis_name="s")


def _fill(ref, n, val, dtype):
    v = jnp.full((16,), val, dtype=dtype)

    def body(i, _):
        ref[pl.ds(i * 16, 16)] = v
        return 0

    lax.fori_loop(0, n // 16, body, 0)


def _fill2d(ref, nrows, w, val, dtype):
    v = jnp.full((16,), val, dtype=dtype)

    def row(r, _):
        def col(j, _):
            ref[r, pl.ds(j * 16, 16)] = v
            return 0

        lax.fori_loop(0, w // 16, col, 0)
        return 0

    lax.fori_loop(0, nrows, row, 0)


def _deg_body(src_hbm, dst_hbm, out_hbm, idxbuf, ones, zbuf, acc_o, acc_i):
    c = lax.axis_index("c")
    s = lax.axis_index("s")
    _fill(zbuf, ZB, 0.0, jnp.float32)
    _fill(ones, CH, 1.0, jnp.float32)
    pltpu.sync_copy(zbuf, acc_o.at[pl.ds(s * ZB, ZB)])
    pltpu.sync_copy(zbuf, acc_i.at[pl.ds(s * ZB, ZB)])
    plsc.subcore_barrier()
    base = (c * 16 + s) * (E2 // 32)

    def chunk(k, _):
        pltpu.sync_copy(src_hbm.at[pl.ds(base + k * CH, CH)], idxbuf)
        pltpu.sync_copy(ones, acc_o.at[idxbuf], add=True)
        pltpu.sync_copy(dst_hbm.at[pl.ds(base + k * CH, CH)], idxbuf)
        pltpu.sync_copy(ones, acc_i.at[idxbuf], add=True)
        return 0

    lax.fori_loop(0, E2 // 32 // CH, chunk, 0)
    plsc.subcore_barrier()
    pltpu.sync_copy(acc_o.at[pl.ds(s * ZB, ZB)], zbuf)
    pltpu.sync_copy(zbuf, out_hbm.at[pl.ds((c * 2 + 0) * NP + s * ZB, ZB)])
    pltpu.sync_copy(acc_i.at[pl.ds(s * ZB, ZB)], zbuf)
    pltpu.sync_copy(zbuf, out_hbm.at[pl.ds((c * 2 + 1) * NP + s * ZB, ZB)])


_deg_call = pl.kernel(
    _deg_body,
    out_type=jax.ShapeDtypeStruct((4 * NP,), jnp.float32),
    mesh=_mesh,
    scratch_types=[
        pltpu.VMEM((CH,), jnp.int32),
        pltpu.VMEM((CH,), jnp.float32),
        pltpu.VMEM((ZB,), jnp.float32),
        pltpu.VMEM_SHARED((NP,), jnp.float32),
        pltpu.VMEM_SHARED((NP,), jnp.float32),
    ],
)


def _s_body(src_hbm, dst_hbm, pk_hbm, out_hbm,
            srcbuf, dstbuf, pkbuf, idxacc, valacc, zbuf, sem, s_sp, pk_sp):
    c = lax.axis_index("c")
    s = lax.axis_index("s")
    _fill(zbuf, ZB, 0.0, jnp.float32)
    for k in range(S_SP // 16 // ZB):
        pltpu.sync_copy(zbuf, s_sp.at[pl.ds(s * (S_SP // 16) + k * ZB, ZB)])
    pltpu.sync_copy(pk_hbm.at[pl.ds(s * ZB, ZB)], zbuf)
    pltpu.sync_copy(zbuf, pk_sp.at[pl.ds(s * ZB, ZB)])
    plsc.subcore_barrier()
    base = s * (E2 // 16)
    lab_lo = c * LW
    lab_hi = LW + c * (LBL - LW)

    def chunk(k, _):
        pltpu.sync_copy(src_hbm.at[pl.ds(base + k * CH, CH)], srcbuf)
        pltpu.sync_copy(dst_hbm.at[pl.ds(base + k * CH, CH)], dstbuf)
        pltpu.async_copy(pk_sp.at[srcbuf], pkbuf, sem).wait()

        def vec(i, _):
            dstv = dstbuf[pl.ds(i * 16, 16)]
            pk = pkbuf[pl.ds(i * 16, 16)]
            labi = lax.shift_right_logical(pk.astype(jnp.int32), 1)
            normv = pk - (labi * 2).astype(jnp.float32)
            inr = (labi >= lab_lo) & (labi < lab_hi) & (dstv < NN)
            flat = dstv * LW + (labi - lab_lo)
            dummy = S_DUMMY + (dstv & 2047)
            idxacc[pl.ds(i * 16, 16)] = jnp.where(inr, flat, dummy)
            valacc[pl.ds(i * 16, 16)] = jnp.where(inr, normv,
                                                  jnp.zeros((16,), jnp.float32))
            return 0

        lax.fori_loop(0, CH // 16, vec, 0)
        pltpu.sync_copy(valacc, s_sp.at[idxacc], add=True)
        return 0

    lax.fori_loop(0, E2 // 16 // CH, chunk, 0)
    plsc.subcore_barrier()

    def cpout(k, _):
        off = s * (S_SP // 16) + k * ZB
        pltpu.sync_copy(s_sp.at[pl.ds(off, ZB)], zbuf)
        pltpu.sync_copy(zbuf, out_hbm.at[pl.ds(c * S_SP + off, ZB)])
        return 0

    lax.fori_loop(0, S_SP // 16 // ZB, cpout, 0)


_s_call = pl.kernel(
    _s_body,
    out_type=jax.ShapeDtypeStruct((2 * S_SP,), jnp.float32),
    mesh=_mesh,
    scratch_types=[
        pltpu.VMEM((CH,), jnp.int32),
        pltpu.VMEM((CH,), jnp.int32),
        pltpu.VMEM((CH,), jnp.float32),
        pltpu.VMEM((CH,), jnp.int32),
        pltpu.VMEM((CH,), jnp.float32),
        pltpu.VMEM((ZB,), jnp.float32),
        pltpu.SemaphoreType.DMA,
        pltpu.VMEM_SHARED((S_SP,), jnp.float32),
        pltpu.VMEM_SHARED((NP,), jnp.float32),
    ],
)


PW = 128
NRG = 4
RNG = NP // NRG
GRP = 80
CHP = 640


def _prop_body(src_hbm, dst_hbm, m_hbm, agg_hbm,
               srcbuf, dstbuf, idx128, loc128, rows, zbuf2, sem, sem2,
               agg_sp):
    ZR = ZB // PW
    rows_per_tile = (RNG + 64) // 16
    c = lax.axis_index("c")
    s = lax.axis_index("s")
    _fill2d(zbuf2, ZR, PW, 0.0, jnp.float32)
    base = s * (E2 // 16)

    for r in range(NRG // 2):
        lo = (c * (NRG // 2) + r) * RNG
        row0 = s * rows_per_tile
        for k in range(rows_per_tile // ZR):
            pltpu.sync_copy(zbuf2, agg_sp.at[pl.ds(row0 + k * ZR, ZR)])
        if rows_per_tile % ZR:
            pltpu.sync_copy(
                zbuf2.at[pl.ds(0, rows_per_tile % ZR)],
                agg_sp.at[pl.ds(row0 + (rows_per_tile // ZR) * ZR,
                                rows_per_tile % ZR)])
        plsc.subcore_barrier()

        NG = CHP // GRP

        def chunk(k, _):
            pltpu.sync_copy(src_hbm.at[pl.ds(base + k * CHP, CHP)], srcbuf)
            pltpu.sync_copy(dst_hbm.at[pl.ds(base + k * CHP, CHP)], dstbuf)

            def vec(i, _):
                off = i * 16
                dstv = dstbuf[pl.ds(off, 16)]
                inr = (dstv >= lo) & (dstv < lo + RNG)
                loc = jnp.where(inr, dstv - lo, RNG + (dstv & 63))
                idx128[i // (GRP // 16), pl.ds((i % (GRP // 16)) * 16, 16)] \
                    = srcbuf[pl.ds(off, 16)]
                loc128[i // (GRP // 16), pl.ds((i % (GRP // 16)) * 16, 16)] \
                    = loc
                return 0

            lax.fori_loop(0, CHP // 16, vec, 0)
            dg = [None] * NG
            ds_ = [None] * NG
            dg[0] = pltpu.async_copy(m_hbm.at[idx128.at[0]], rows.at[0], sem)
            for g in range(NG):
                b = g % 2
                dg[g].wait()
                if g + 1 < NG:
                    if g >= 1:
                        ds_[g - 1].wait()
                    dg[g + 1] = pltpu.async_copy(
                        m_hbm.at[idx128.at[g + 1]], rows.at[(g + 1) % 2],
                        sem)
                ds_[g] = pltpu.async_copy(rows.at[b],
                                          agg_sp.at[loc128.at[g]], sem2,
                                          add=True)
            ds_[NG - 2].wait()
            ds_[NG - 1].wait()
            return 0

        lax.fori_loop(0, E2 // 16 // CHP, chunk, 0)
        plsc.subcore_barrier()
        out_rows = RNG // 16
        for k in range(out_rows // GRP):
            pltpu.sync_copy(agg_sp.at[pl.ds(s * out_rows + k * GRP, GRP)],
                            rows.at[0])
            pltpu.sync_copy(
                rows.at[0],
                agg_hbm.at[pl.ds(lo + s * out_rows + k * GRP, GRP)])
        orem = out_rows % GRP
        if orem:
            pltpu.sync_copy(
                agg_sp.at[pl.ds(s * out_rows + (out_rows // GRP) * GRP,
                                orem)],
                rows.at[0, pl.ds(0, orem)])
            pltpu.sync_copy(
                rows.at[0, pl.ds(0, orem)],
                agg_hbm.at[pl.ds(lo + s * out_rows + (out_rows // GRP)
                                 * GRP, orem)])
        plsc.subcore_barrier()


_prop = pl.kernel(
    _prop_body,
    out_type=jax.ShapeDtypeStruct((NP, PW), jnp.float32),
    mesh=_mesh,
    scratch_types=[
        pltpu.VMEM((CHP,), jnp.int32),
        pltpu.VMEM((CHP,), jnp.int32),
        pltpu.VMEM((CHP // GRP, GRP), jnp.int32),
        pltpu.VMEM((CHP // GRP, GRP), jnp.int32),
        pltpu.VMEM((2, GRP, PW), jnp.float32),
        pltpu.VMEM((ZB // PW, PW), jnp.float32),
        pltpu.SemaphoreType.DMA,
        pltpu.SemaphoreType.DMA,
        pltpu.VMEM_SHARED((RNG + 64, PW), jnp.float32),
    ],
)


def _norm_body(deg_ref, lab_ref, ns_ref, nd_ref, pk_ref):
    d = deg_ref[...]
    deg_o = d[0, 0] + d[1, 0]
    deg_i = d[0, 1] + d[1, 1]
    ns = lax.rsqrt(jnp.maximum(deg_o, 1.0))
    nd = lax.rsqrt(jnp.maximum(deg_i, 1.0))
    ns_ref[...] = ns[:, None]
    nd_ref[...] = nd[:, None]
    pk_ref[...] = ns + 2.0 * lab_ref[..., 0]


def _t1_body(emb_ref, w1_ref, t1_ref):
    t1_ref[...] = jnp.dot(emb_ref[...], w1_ref[...],
                          preferred_element_type=jnp.float32)


def _layer1_body(s0_ref, s1_ref, emb_ref, w1_ref, w2_ref, ns_ref, nd_ref,
                 b1_ref, m2_ref):
    t1 = jnp.dot(emb_ref[...], w1_ref[...], preferred_element_type=jnp.float32)
    agg = (jnp.dot(s0_ref[...], t1[0:LW], preferred_element_type=jnp.float32)
           + jnp.dot(s1_ref[...], t1[LW:2 * LW],
                     preferred_element_type=jnp.float32))
    hn = jax.nn.relu(agg * nd_ref[...] + b1_ref[...]) * ns_ref[...]
    m2_ref[...] = jnp.dot(hn, w2_ref[...], preferred_element_type=jnp.float32)


def _layer2_body(agg_ref, w3_ref, ns_ref, nd_ref, b2_ref, m3_ref):
    h = jax.nn.relu(agg_ref[...][:, :100] * nd_ref[...] + b2_ref[...])
    m3_ref[...] = jnp.dot(h * ns_ref[...], w3_ref[...],
                          preferred_element_type=jnp.float32)


def _layer3_body(agg_ref, nd_ref, b3_ref, out_ref):
    out_ref[...] = agg_ref[...][:, :LBL] * nd_ref[...] + b3_ref[...]


_BLK = 512
_GRID = NP // _BLK


def _col_spec(w):
    return pl.BlockSpec((_BLK, w), lambda i: (i, 0))


def _full_spec(r, w):
    return pl.BlockSpec((r, w), lambda i: (0, 0))


def kernel(dep_labels, edge_index, emb, W1, b1, W2, b2, W3, b3):
    f32 = jnp.float32
    pad_ids = PB + (jnp.arange(E2 - EE, dtype=jnp.int32) % 128)
    src_p = jnp.concatenate([edge_index[0], pad_ids])
    dst_p = jnp.concatenate([edge_index[1], pad_ids])
    lab1d = jnp.pad(dep_labels, (0, NP - NN))
    embp = jnp.pad(emb, ((0, 1), (0, 14)))
    W1p = jnp.pad(W1, ((0, 14), (0, 12)))
    W2p = jnp.pad(W2, ((0, 12), (0, 28)))
    W3p = jnp.pad(W3, ((0, 0), (0, 81)))
    b1p = jnp.pad(b1, (0, 12))[None, :]
    b2p = b2[None, :]
    b3p = b3[None, :]

    degpart = _deg_call(src_p, dst_p).reshape(2, 2, NP)

    ns2d, nd2d, packed = pl.pallas_call(
        _norm_body,
        grid=(_GRID,),
        in_specs=[pl.BlockSpec((2, 2, _BLK), lambda i: (0, 0, i)),
                  _col_spec(1)],
        out_specs=[_col_spec(1), _col_spec(1),
                   pl.BlockSpec((_BLK,), lambda i: (i,))],
        out_shape=[jax.ShapeDtypeStruct((NP, 1), f32),
                   jax.ShapeDtypeStruct((NP, 1), f32),
                   jax.ShapeDtypeStruct((NP,), f32)],
    )(degpart, lab1d.astype(f32)[:, None])

    s_flat = _s_call(src_p, dst_p, packed).reshape(2, NP, LW)
    s0 = s_flat[0]
    s1 = s_flat[1]

    m2 = pl.pallas_call(
        _layer1_body,
        grid=(_GRID,),
        in_specs=[_col_spec(LW), _col_spec(LW), _full_spec(48, 64),
                  _full_spec(64, 112), _full_spec(112, PW), _col_spec(1),
                  _col_spec(1), _full_spec(1, 112)],
        out_specs=_col_spec(PW),
        out_shape=jax.ShapeDtypeStruct((NP, PW), f32),
    )(s0, s1, embp, W1p, W2p, ns2d, nd2d, b1p)

    agg2 = _prop(src_p, dst_p, m2)

    m3 = pl.pallas_call(
        _layer2_body,
        grid=(_GRID,),
        in_specs=[_col_spec(PW), _full_spec(100, PW),
                  _col_spec(1), _col_spec(1), _full_spec(1, 100)],
        out_specs=_col_spec(PW),
        out_shape=jax.ShapeDtypeStruct((NP, PW), f32),
    )(agg2, W3p, ns2d, nd2d, b2p)

    agg3 = _prop(src_p, dst_p, m3)

    out = pl.pallas_call(
        _layer3_body,
        grid=(_GRID,),
        in_specs=[_col_spec(PW), _col_spec(1), _full_spec(1, LBL)],
        out_specs=_col_spec(LBL),
        out_shape=jax.ShapeDtypeStruct((NP, LBL), f32),
    )(agg3, nd2d, b3p)

    return out[:NN]
